# baseline (speedup 1.0000x reference)
import functools

import jax
import jax.numpy as jnp
from jax import lax
from jax.experimental import pallas as pl
from jax.experimental.pallas import tpu as pltpu

LANES = 128


def _prep2_mats():
    k2 = jnp.arange(128)[:, None]
    col = jnp.arange(256)[None, :]
    b, l = col // 128, col % 128
    s2m = ((k2 == 2 * (l % 64) + b - 2) & (k2 < 74)).astype(jnp.bfloat16)
    row = jnp.arange(96)[:, None]
    s, ar = row // 48, row % 48
    a, r2 = ar // 24, ar % 24
    j = jnp.arange(74)[None, :]
    rsm = ((j == 4 * r2 + a - 2 + 2 * s) & (r2 < 21)).astype(jnp.bfloat16)
    return s2m, rsm


def _prep2_body(y_ref, s2m_ref, rsm_ref, o_ref):
    s2m, rsm = s2m_ref[...], rsm_ref[...]
    lane = lax.broadcasted_iota(jnp.int32, (21, 256), 1) % 128
    for img in range(y_ref.shape[0]):
        y1 = y_ref[img]
        for ci in range(16):
            v = jnp.dot(y1[ci], s2m,
                        preferred_element_type=jnp.float32).astype(jnp.bfloat16)
            g = jnp.dot(rsm, v,
                        preferred_element_type=jnp.float32).astype(jnp.bfloat16)
            for a in range(2):
                pk = jnp.where(lane < 64, g[a * 24:a * 24 + 21],
                               g[48 + a * 24:48 + a * 24 + 21])
                o_ref[img, a * 32 + ci] = pk[:, 0:128]
                o_ref[img, a * 32 + 16 + ci] = pk[:, 128:256]


def _conv1_body(x_ref, w_ref, s_ref, o_ref, xs_ref):
    l_out = 74 * LANES
    nb = x_ref.shape[0]
    for img in range(nb):
        for r in range(3):
            for c in range(3):
                off = r * LANES + c
                t = 3 * r + c
                xs_ref[t * 16:(t + 1) * 16, img * l_out:(img + 1) * l_out] = (
                    x_ref[img, :, off:off + l_out])
    acc = jnp.dot(w_ref[...], xs_ref[...], preferred_element_type=jnp.float32)
    z = jnp.maximum(acc + s_ref[...], 0.0)
    y = jnp.maximum(jnp.maximum(z[:16], z[16:32]),
                    jnp.maximum(z[32:48], z[48:64]))
    lane = lax.broadcasted_iota(jnp.int32, y.shape, 1) % LANES
    y = jnp.where(lane < 74, y, 0.0).astype(jnp.bfloat16)
    for img in range(nb):
        o_ref[img] = y[:, img * l_out:(img + 1) * l_out]


def _conv2_fc_body(x_ref, w_ref, s_ref, wfc_ref, o_ref, xs_ref):
    l_out = 19 * LANES
    nb = x_ref.shape[0]
    for img in range(nb):
        for r in range(3):
            for c in range(3):
                off = r * 64 + c
                t = 3 * r + c
                xs_ref[t * 64:(t + 1) * 64, img * l_out:(img + 1) * l_out] = (
                    x_ref[img, :, off:off + l_out])
    acc = jnp.dot(w_ref[...], xs_ref[...], preferred_element_type=jnp.float32)
    z = jnp.maximum(acc + s_ref[...], 0.0)
    y2 = jnp.maximum(jnp.maximum(z[:32], z[32:64]),
                     jnp.maximum(z[64:96], z[96:128]))
    wfc = wfc_ref[...]
    for img in range(nb):
        o_ref[img, 0] = jnp.sum(
            wfc * y2[None, :, img * l_out:(img + 1) * l_out], axis=(1, 2))


def _prep1(x):
    n = x.shape[0]
    x = x.astype(jnp.bfloat16)
    x = jnp.pad(x, ((0, 0), (0, 1), (2, 2), (2, 2)))
    x = x.reshape(n, 4, 76, 2, 76, 2).transpose(0, 3, 5, 1, 2, 4)
    x = x.reshape(n, 16, 76, 76)
    x = jnp.pad(x, ((0, 0), (0, 0), (0, 1), (0, LANES - 76)))
    return x.reshape(n, 16, 77 * LANES)


def _prep2(y1):
    n = y1.shape[0]
    y = jnp.pad(y1, ((0, 0), (0, 0), (2, 2), (2, 0)))[..., :LANES]
    y = y.reshape(n, 16, 39, 2, 64, 2).transpose(0, 3, 5, 1, 2, 4)
    y = y.reshape(n, 64, 39, 64)
    y = jnp.pad(y, ((0, 0), (0, 0), (0, 1), (0, 0)))
    y = y.reshape(n, 64, 20, LANES)
    y = jnp.pad(y, ((0, 0), (0, 0), (0, 1), (0, 0)))
    return y.reshape(n, 64, 21 * LANES)


@jax.jit
def kernel(x, w1, s1, w2, s2, wfc, bfc):
    n = x.shape[0]
    y1 = pl.pallas_call(
        _conv1_body,
        out_shape=jax.ShapeDtypeStruct((n, 16, 74 * LANES), jnp.bfloat16),
        grid=(n // 4,),
        in_specs=[pl.BlockSpec((4, 16, 77 * LANES), lambda i: (i, 0, 0)),
                  pl.BlockSpec((64, 144), lambda i: (0, 0)),
                  pl.BlockSpec((64, 1), lambda i: (0, 0))],
        out_specs=pl.BlockSpec((4, 16, 74 * LANES), lambda i: (i, 0, 0)),
        scratch_shapes=[pltpu.VMEM((144, 4 * 74 * LANES), jnp.bfloat16)],
        compiler_params=pltpu.CompilerParams(
            dimension_semantics=("parallel",),
            vmem_limit_bytes=64 * 1024 * 1024),
    )(_prep1(x), w1, s1)

    s2m, rsm = _prep2_mats()
    packed = pl.pallas_call(
        _prep2_body,
        out_shape=jax.ShapeDtypeStruct((n, 64, 21, 128), jnp.bfloat16),
        grid=(n // 4,),
        in_specs=[pl.BlockSpec((4, 16, 74, 128), lambda i: (i, 0, 0, 0)),
                  pl.BlockSpec((128, 256), lambda i: (0, 0)),
                  pl.BlockSpec((96, 74), lambda i: (0, 0))],
        out_specs=pl.BlockSpec((4, 64, 21, 128), lambda i: (i, 0, 0, 0)),
        compiler_params=pltpu.CompilerParams(
            dimension_semantics=("parallel",),
            vmem_limit_bytes=64 * 1024 * 1024),
    )(y1.reshape(n, 16, 74, 128), s2m, rsm)

    logits = pl.pallas_call(
        _conv2_fc_body,
        out_shape=jax.ShapeDtypeStruct((n, 1, 6), jnp.float32),
        grid=(n // 4,),
        in_specs=[pl.BlockSpec((4, 64, 21 * LANES), lambda i: (i, 0, 0)),
                  pl.BlockSpec((128, 576), lambda i: (0, 0)),
                  pl.BlockSpec((128, 1), lambda i: (0, 0)),
                  pl.BlockSpec((6, 32, 19 * LANES), lambda i: (0, 0, 0))],
        out_specs=pl.BlockSpec((4, 1, 6), lambda i: (i, 0, 0)),
        scratch_shapes=[pltpu.VMEM((576, 4 * 19 * LANES), jnp.bfloat16)],
        compiler_params=pltpu.CompilerParams(
            dimension_semantics=("parallel",),
            vmem_limit_bytes=64 * 1024 * 1024),
    )(packed.reshape(n, 64, 21 * LANES), w2, s2,
      wfc.reshape(6, 32, 19 * LANES))

    return logits.reshape(n, 6) + bfc[None, :]


# speedup vs baseline: 1.2538x; 1.2538x over previous
import functools

import jax
import jax.numpy as jnp
from jax import lax
from jax.experimental import pallas as pl
from jax.experimental.pallas import tpu as pltpu

LANES = 128


def _conv1_body(x_ref, w_ref, s_ref, o_ref, xs_ref):
    l_out = 74 * LANES
    nb = x_ref.shape[0]
    for img in range(nb):
        for r in range(3):
            for c in range(3):
                off = r * LANES + c
                t = 3 * r + c
                xs_ref[t * 16:(t + 1) * 16, img * l_out:(img + 1) * l_out] = (
                    x_ref[img, :, off:off + l_out])
    acc = jnp.dot(w_ref[...], xs_ref[...], preferred_element_type=jnp.float32)
    z = jnp.maximum(acc + s_ref[...], 0.0)
    y = jnp.maximum(jnp.maximum(z[:16], z[16:32]),
                    jnp.maximum(z[32:48], z[48:64]))
    lane = lax.broadcasted_iota(jnp.int32, y.shape, 1) % LANES
    y = jnp.where(lane < 74, y, 0.0).astype(jnp.bfloat16)
    for img in range(nb):
        o_ref[img] = y[:, img * l_out:(img + 1) * l_out]


def _conv2_fc_body(x_ref, w_ref, s_ref, wfc_ref, o_ref, xs_ref):
    l_out = 19 * LANES
    nb = x_ref.shape[0]
    for img in range(nb):
        for r in range(3):
            for c in range(3):
                off = r * 64 + c
                t = 3 * r + c
                xs_ref[t * 64:(t + 1) * 64, img * l_out:(img + 1) * l_out] = (
                    x_ref[img, :, off:off + l_out])
    acc = jnp.dot(w_ref[...], xs_ref[...], preferred_element_type=jnp.float32)
    z = jnp.maximum(acc + s_ref[...], 0.0)
    y2 = jnp.maximum(jnp.maximum(z[:32], z[32:64]),
                     jnp.maximum(z[64:96], z[96:128]))
    wfc = wfc_ref[...]
    for img in range(nb):
        o_ref[img, 0] = jnp.sum(
            wfc * y2[None, :, img * l_out:(img + 1) * l_out], axis=(1, 2))


def _prep1(x):
    n = x.shape[0]
    x = x.astype(jnp.bfloat16)
    x = jnp.pad(x, ((0, 0), (0, 1), (2, 2), (2, 2)))
    x = x.reshape(n, 4, 76, 2, 76, 2).transpose(0, 3, 5, 1, 2, 4)
    x = x.reshape(n, 16, 76, 76)
    x = jnp.pad(x, ((0, 0), (0, 0), (0, 1), (0, LANES - 76)))
    return x.reshape(n, 16, 77 * LANES)


def _prep2(y1):
    n = y1.shape[0]
    y = jnp.pad(y1, ((0, 0), (0, 0), (2, 2), (2, 0)))[..., :LANES]
    y = y.reshape(n, 16, 39, 2, 64, 2).transpose(0, 3, 5, 1, 2, 4)
    y = y.reshape(n, 64, 39, 64)
    y = jnp.pad(y, ((0, 0), (0, 0), (0, 1), (0, 0)))
    y = y.reshape(n, 64, 20, LANES)
    y = jnp.pad(y, ((0, 0), (0, 0), (0, 1), (0, 0)))
    return y.reshape(n, 64, 21 * LANES)


@jax.jit
def kernel(x, w1, s1, w2, s2, wfc, bfc):
    n = x.shape[0]
    y1 = pl.pallas_call(
        _conv1_body,
        out_shape=jax.ShapeDtypeStruct((n, 16, 74 * LANES), jnp.bfloat16),
        grid=(n // 8,),
        in_specs=[pl.BlockSpec((8, 16, 77 * LANES), lambda i: (i, 0, 0)),
                  pl.BlockSpec((64, 144), lambda i: (0, 0)),
                  pl.BlockSpec((64, 1), lambda i: (0, 0))],
        out_specs=pl.BlockSpec((8, 16, 74 * LANES), lambda i: (i, 0, 0)),
        scratch_shapes=[pltpu.VMEM((144, 8 * 74 * LANES), jnp.bfloat16)],
        compiler_params=pltpu.CompilerParams(
            dimension_semantics=("parallel",),
            vmem_limit_bytes=64 * 1024 * 1024),
    )(_prep1(x), w1, s1)

    logits = pl.pallas_call(
        _conv2_fc_body,
        out_shape=jax.ShapeDtypeStruct((n, 1, 6), jnp.float32),
        grid=(n // 8,),
        in_specs=[pl.BlockSpec((8, 64, 21 * LANES), lambda i: (i, 0, 0)),
                  pl.BlockSpec((128, 576), lambda i: (0, 0)),
                  pl.BlockSpec((128, 1), lambda i: (0, 0)),
                  pl.BlockSpec((6, 32, 19 * LANES), lambda i: (0, 0, 0))],
        out_specs=pl.BlockSpec((8, 1, 6), lambda i: (i, 0, 0)),
        scratch_shapes=[pltpu.VMEM((576, 8 * 19 * LANES), jnp.bfloat16)],
        compiler_params=pltpu.CompilerParams(
            dimension_semantics=("parallel",),
            vmem_limit_bytes=64 * 1024 * 1024),
    )(_prep2(y1.reshape(n, 16, 74, LANES)), w2, s2,
      wfc.reshape(6, 32, 19 * LANES))

    return logits.reshape(n, 6) + bfc[None, :]
